# baseline (speedup 1.0000x reference)
import functools

import jax
import jax.numpy as jnp
from jax import lax
from jax.experimental import pallas as pl
from jax.experimental.pallas import tpu as pltpu
from jax.experimental.pallas import tpu_sc as plsc

FEAT = 512
BATCH = 16384
NC, NS, L = 2, 16, 16
NW = NC * NS

SCB = 6144
TCB = BATCH - SCB

BPW3 = SCB // NW
C3 = 32
NCHUNK3 = BPW3 // C3
NPAIR3 = NCHUNK3 // 2

BPW1 = TCB // NW
C1 = 40
NCHUNK1 = BPW1 // C1
NSLOT1 = 4

RB = 512
TC_OFF = SCB // RB


def _sc_full_body(mu_hbm, lv_hbm, lab_hbm, fcw_hbm, out_hbm,
                  idx_v, g0, m0, l0, g1, m1, l1, acc_v, sem0, sem1):
    wid = lax.axis_index("s") * NC + lax.axis_index("c")
    base = wid * BPW3
    pltpu.sync_copy(lab_hbm.at[pl.ds(base, BPW3)], idx_v)

    def issue(k, g, m, l, sem):
        row0 = base + k * C3
        pltpu.async_copy(fcw_hbm.at[idx_v.at[pl.ds(k * C3, C3)]], g, sem)
        pltpu.async_copy(mu_hbm.at[pl.ds(row0, C3)], m, sem)
        pltpu.async_copy(lv_hbm.at[pl.ds(row0, C3)], l, sem)

    def drain(k, g, m, l, sem):
        row0 = base + k * C3
        pltpu.make_async_copy(fcw_hbm.at[idx_v.at[pl.ds(k * C3, C3)]], g, sem).wait()
        pltpu.make_async_copy(mu_hbm.at[pl.ds(row0, C3)], m, sem).wait()
        pltpu.make_async_copy(lv_hbm.at[pl.ds(row0, C3)], l, sem).wait()

    def consume(g_v, mu_v, lv_v, acc):
        def row(r, acc):
            af, av = acc
            for c in range(FEAT // L):
                sl = pl.ds(c * L, L)
                g = g_v[r, sl]
                m = mu_v[r, sl]
                v = lv_v[r, sl]
                d = g - m
                af = af + (d * d) * jnp.exp(-v)
                av = av + v
            return af, av

        return lax.fori_loop(0, C3, row, acc)

    issue(0, g0, m0, l0, sem0)
    issue(1, g1, m1, l1, sem1)

    def pair(p, acc):
        k0 = 2 * p
        not_last = p < NPAIR3 - 1
        drain(k0, g0, m0, l0, sem0)
        acc = consume(g0, m0, l0, acc)

        @pl.when(not_last)
        def _():
            issue(k0 + 2, g0, m0, l0, sem0)

        drain(k0 + 1, g1, m1, l1, sem1)
        acc = consume(g1, m1, l1, acc)

        @pl.when(not_last)
        def _():
            issue(k0 + 3, g1, m1, l1, sem1)

        return acc

    zero = jnp.zeros((L,), jnp.float32)
    af, av = lax.fori_loop(0, NPAIR3, pair, (zero, zero))

    acc_v[...] = af + av
    pltpu.sync_copy(acc_v, out_hbm.at[wid])


def _sc_gather_body(lab_hbm, fcw_hbm, cen_hbm, idx_v,
                    b0, b1, b2, b3, gs0, gs1, gs2, gs3, ss0, ss1, ss2, ss3):
    wid = lax.axis_index("s") * NC + lax.axis_index("c")
    base = wid * BPW1
    pltpu.sync_copy(lab_hbm.at[pl.ds(SCB + base, BPW1)], idx_v)

    bufs = (b0, b1, b2, b3)
    gsems = (gs0, gs1, gs2, gs3)
    ssems = (ss0, ss1, ss2, ss3)

    def g_copy(k):
        s = k % NSLOT1
        return pltpu.make_async_copy(
            fcw_hbm.at[idx_v.at[pl.ds(k * C1, C1)]], bufs[s], gsems[s])

    def s_copy(k):
        s = k % NSLOT1
        return pltpu.make_async_copy(
            bufs[s], cen_hbm.at[pl.ds(base + k * C1, C1)], ssems[s])

    for k in range(NSLOT1):
        g_copy(k).start()
    for k in range(NCHUNK1):
        g_copy(k).wait()
        s_copy(k).start()
        if k + NSLOT1 < NCHUNK1:
            s_copy(k).wait()
            g_copy(k + NSLOT1).start()
    for k in range(NCHUNK1 - NSLOT1, NCHUNK1):
        s_copy(k).wait()


def _tc_body(c_ref, m_ref, v_ref, o_ref):
    i = pl.program_id(0)

    @pl.when(i == 0)
    def _():
        o_ref[...] = jnp.zeros_like(o_ref)

    d = c_ref[...] - m_ref[...]
    t = (d * d) * jnp.exp(-v_ref[...]) + v_ref[...]
    o_ref[...] += jnp.sum(t.reshape(RB * (FEAT // 128), 128),
                          axis=0, keepdims=True)


def kernel(mu, logvar, labels, fc_weights):
    labels = labels.astype(jnp.int32)
    mesh = plsc.VectorSubcoreMesh(
        core_axis_name="c", subcore_axis_name="s",
        num_cores=NC, num_subcores=NS)

    cbuf = lambda: pltpu.VMEM((C1, FEAT), jnp.float32)
    centers = pl.kernel(
        _sc_gather_body,
        out_type=jax.ShapeDtypeStruct((TCB, FEAT), jnp.float32),
        mesh=mesh,
        scratch_types=[
            pltpu.VMEM((BPW1,), jnp.int32),
            cbuf(), cbuf(), cbuf(), cbuf(),
            pltpu.SemaphoreType.DMA, pltpu.SemaphoreType.DMA,
            pltpu.SemaphoreType.DMA, pltpu.SemaphoreType.DMA,
            pltpu.SemaphoreType.DMA, pltpu.SemaphoreType.DMA,
            pltpu.SemaphoreType.DMA, pltpu.SemaphoreType.DMA,
        ],
    )(labels, fc_weights)

    buf = lambda: pltpu.VMEM((C3, FEAT), jnp.float32)
    sc_part = pl.kernel(
        _sc_full_body,
        out_type=jax.ShapeDtypeStruct((NW, L), jnp.float32),
        mesh=mesh,
        scratch_types=[
            pltpu.VMEM((BPW3,), jnp.int32),
            buf(), buf(), buf(), buf(), buf(), buf(),
            pltpu.VMEM((L,), jnp.float32),
            pltpu.SemaphoreType.DMA,
            pltpu.SemaphoreType.DMA,
        ],
    )(mu, logvar, labels, fc_weights)

    tc_part = pl.pallas_call(
        _tc_body,
        grid=(TCB // RB,),
        in_specs=[
            pl.BlockSpec((RB, FEAT), lambda i: (i, 0)),
            pl.BlockSpec((RB, FEAT), lambda i: (i + TC_OFF, 0)),
            pl.BlockSpec((RB, FEAT), lambda i: (i + TC_OFF, 0)),
        ],
        out_specs=pl.BlockSpec((1, 128), lambda i: (0, 0)),
        out_shape=jax.ShapeDtypeStruct((1, 128), jnp.float32),
    )(centers, mu, logvar)

    return (jnp.sum(sc_part) + jnp.sum(tc_part)) / (2.0 * BATCH)


# speedup vs baseline: 1.2263x; 1.2263x over previous
import functools

import jax
import jax.numpy as jnp
from jax import lax
from jax.experimental import pallas as pl
from jax.experimental.pallas import tpu as pltpu
from jax.experimental.pallas import tpu_sc as plsc

FEAT = 512
BATCH = 16384
NC, NS, L = 2, 16, 16
NW = NC * NS
BPW = BATCH // NW
C = 16
NCHUNK = BPW // C
NSLOT = 4
NGROUP = NCHUNK // NSLOT


def _sc_body(mu_hbm, lv_hbm, lab_hbm, fcw_hbm, out_hbm, idx_v,
             g0, m0, l0, g1, m1, l1, g2, m2, l2, g3, m3, l3,
             acc_v, sem0, sem1, sem2, sem3):
    wid = lax.axis_index("s") * NC + lax.axis_index("c")
    base = wid * BPW
    pltpu.sync_copy(lab_hbm.at[pl.ds(base, BPW)], idx_v)

    def issue(k, g, m, l, sem):
        row0 = base + k * C
        pltpu.async_copy(fcw_hbm.at[idx_v.at[pl.ds(k * C, C)]], g, sem)
        pltpu.async_copy(mu_hbm.at[pl.ds(row0, C)], m, sem)
        pltpu.async_copy(lv_hbm.at[pl.ds(row0, C)], l, sem)

    def drain(k, g, m, l, sem):
        row0 = base + k * C
        pltpu.make_async_copy(fcw_hbm.at[idx_v.at[pl.ds(k * C, C)]], g, sem).wait()
        pltpu.make_async_copy(mu_hbm.at[pl.ds(row0, C)], m, sem).wait()
        pltpu.make_async_copy(lv_hbm.at[pl.ds(row0, C)], l, sem).wait()

    def consume(g_v, mu_v, lv_v, acc):
        def row(r, acc):
            af, av = acc
            for c in range(FEAT // L):
                sl = pl.ds(c * L, L)
                g = g_v[r, sl]
                m = mu_v[r, sl]
                v = lv_v[r, sl]
                d = g - m
                af = af + (d * d) * jnp.exp(-v)
                av = av + v
            return af, av

        return lax.fori_loop(0, C, row, acc)

    slots = ((g0, m0, l0, sem0), (g1, m1, l1, sem1),
             (g2, m2, l2, sem2), (g3, m3, l3, sem3))

    for j in range(NSLOT):
        issue(j, *slots[j])

    def group(gi, acc):
        not_last = gi < NGROUP - 1
        for j in range(NSLOT):
            k = gi * NSLOT + j
            drain(k, *slots[j])
            acc = consume(*slots[j][:3], acc)

            @pl.when(not_last)
            def _(k=k, j=j):
                issue(k + NSLOT, *slots[j])

        return acc

    zero = jnp.zeros((L,), jnp.float32)
    af, av = lax.fori_loop(0, NGROUP, group, (zero, zero))

    acc_v[...] = af + av
    pltpu.sync_copy(acc_v, out_hbm.at[wid])


def kernel(mu, logvar, labels, fc_weights):
    labels = labels.astype(jnp.int32)
    mesh = plsc.VectorSubcoreMesh(
        core_axis_name="c", subcore_axis_name="s",
        num_cores=NC, num_subcores=NS)
    buf = lambda: pltpu.VMEM((C, FEAT), jnp.float32)
    partials = pl.kernel(
        _sc_body,
        out_type=jax.ShapeDtypeStruct((NW, L), jnp.float32),
        mesh=mesh,
        scratch_types=[
            pltpu.VMEM((BPW,), jnp.int32),
            buf(), buf(), buf(), buf(), buf(), buf(),
            buf(), buf(), buf(), buf(), buf(), buf(),
            pltpu.VMEM((L,), jnp.float32),
            pltpu.SemaphoreType.DMA,
            pltpu.SemaphoreType.DMA,
            pltpu.SemaphoreType.DMA,
            pltpu.SemaphoreType.DMA,
        ],
    )(mu, logvar, labels, fc_weights)
    return jnp.sum(partials) / (2.0 * BATCH)
